# speedup vs baseline: 3.2498x; 3.2498x over previous
import jax
import jax.numpy as jnp
from jax.experimental import pallas as pl
from jax.experimental.pallas import tpu as pltpu

_B, _C, _H, _W = 4, 384, 32, 32
_HW = _H * _W
_H2, _W2 = 16, 16
_HW2 = _H2 * _W2

_HI = jax.lax.Precision.HIGHEST


def _mu_kernel(fp_ref, fms_ref, rt_ref, out_ref):
    rt = rt_ref[...]
    iota_r = jax.lax.broadcasted_iota(jnp.int32, (_C, _C), 0)
    iota_c = jax.lax.broadcasted_iota(jnp.int32, (_C, _C), 1)

    sums_all = []
    listk_all = []
    for b in range(_B):
        fp = fp_ref[b]
        fms = fms_ref[b]
        ms_res = jax.lax.dot_general(
            fms, rt, (((1,), (0,)), ((), ())),
            precision=_HI, preferred_element_type=jnp.float32)
        ms_d = ms_res - jnp.mean(ms_res, axis=1, keepdims=True)
        p_d = fp - jnp.mean(fp, axis=1, keepdims=True)
        p_n = jnp.sqrt(jnp.sum(p_d * p_d, axis=1, keepdims=True))
        m_n = jnp.sqrt(jnp.sum(ms_d * ms_d, axis=1, keepdims=True))
        denom = m_n * p_n * 0.01
        s = jax.lax.dot_general(
            ms_d, p_d, (((1,), (1,)), ((), ())),
            precision=_HI, preferred_element_type=jnp.float32) / denom
        mv = jnp.max(s, axis=1, keepdims=True)
        idx = jnp.min(jnp.where(s == mv, iota_c, _C), axis=1, keepdims=True)
        e = jnp.exp(mv - jnp.max(mv))
        max_val = e / jnp.sum(e)
        hits = idx == iota_c
        sums = jnp.sum(jnp.where(hits, max_val, 0.0), axis=0, keepdims=True)
        present = jnp.sum(hits.astype(jnp.int32), axis=0, keepdims=True) > 0
        listk = jnp.sum(present.astype(jnp.int32))
        sums_all.append(sums)
        listk_all.append(listk)

    min_lk = listk_all[0]
    for b in range(1, _B):
        min_lk = jnp.minimum(min_lk, listk_all[b])
    min_k = (min_lk + 1) // 2

    for b in range(_B):
        sums = sums_all[b]
        srow = sums
        scol = jnp.transpose(sums)
        before = (srow > scol) | ((srow == scol) & (iota_c > iota_r))
        rank = jnp.sum(before.astype(jnp.int32), axis=1, keepdims=True)
        selm = rank < min_k
        mx = jnp.max(sums)
        w = jnp.where(selm, jnp.exp(jnp.transpose(sums) - mx), 0.0)
        w = w / jnp.sum(w)
        fp = fp_ref[b]
        sig = jax.nn.sigmoid(fp)
        maskv = jax.lax.dot_general(
            jnp.transpose(w), sig, (((1,), (0,)), ((), ())),
            precision=_HI, preferred_element_type=jnp.float32)
        out_ref[b] = fp * (1.0 + maskv)


def kernel(f_p, f_ms, interpret=False):
    B, C, H, W = f_p.shape
    a = jax.image.resize(jnp.eye(_H2, dtype=jnp.float32), (_H, _H2),
                         method="bilinear")
    rt = jnp.transpose(jnp.kron(a, a))
    fp_flat = f_p.reshape(B, C, H * W)
    fms_flat = f_ms.reshape(B, C, _HW2)
    out = pl.pallas_call(
        _mu_kernel,
        out_shape=jax.ShapeDtypeStruct((B, C, H * W), jnp.float32),
        in_specs=[
            pl.BlockSpec(memory_space=pltpu.VMEM),
            pl.BlockSpec(memory_space=pltpu.VMEM),
            pl.BlockSpec(memory_space=pltpu.VMEM),
        ],
        out_specs=pl.BlockSpec(memory_space=pltpu.VMEM),
        interpret=interpret,
    )(fp_flat, fms_flat, rt)
    return out.reshape(B, C, H, W)
